# Initial kernel scaffold; baseline (speedup 1.0000x reference)
import jax
import jax.numpy as jnp
from jax.experimental import pallas as pl


def kernel(xyz, feature, raw_neighbors_feature, neighbors_idx_before, sample_indexes, Ww, bw, gw, betw, Wb, bb, gb, betb, Wo, bo, go, beto):
    raise NotImplementedError("write your pallas kernel here")


# speedup vs baseline: 1.9622x; 1.9622x over previous
import functools
from itertools import combinations

import jax
import jax.numpy as jnp
from jax.experimental import pallas as pl

A = 4
K = 16
INTRA = A * (A - 1) // 2
CH = 2 * INTRA + A * A
PAIRS = list(combinations(range(1, A), 2))

_INF = float("inf")
_HI = jax.lax.Precision.HIGHEST


def _dot(a, b):
    return jnp.dot(a, b, preferred_element_type=jnp.float32, precision=_HI)


def _onehot(idx_col, lane_iota):
    return jnp.where(lane_iota == idx_col, 1.0, 0.0).astype(jnp.float32)


def _selectors(f32):
    pq_col = jax.lax.broadcasted_iota(jnp.int32, (A, A * A), 1)
    pq_row = jax.lax.broadcasted_iota(jnp.int32, (A, A * A), 0)
    Psel = jnp.where(pq_col // A == pq_row, 1.0, 0.0).astype(f32)
    Qsel = jnp.where(pq_col % A == pq_row, 1.0, 0.0).astype(f32)
    pr_row = jax.lax.broadcasted_iota(jnp.int32, (A, len(PAIRS)), 0)
    pr_col = jax.lax.broadcasted_iota(jnp.int32, (A, len(PAIRS)), 1)
    E1 = jnp.zeros((A, len(PAIRS)), f32)
    E2 = jnp.zeros((A, len(PAIRS)), f32)
    for j, (p, q) in enumerate(PAIRS):
        hit = pr_col == j
        E1 = jnp.where(hit & (pr_row == p), 1.0, E1).astype(f32)
        E2 = jnp.where(hit & (pr_row == q), 1.0, E2).astype(f32)
    return Psel, Qsel, E1, E2


def _sel_dists(cols_a, cols_b, sel_a, sel_b):
    acc = None
    for ca, cb in zip(cols_a, cols_b):
        diff = _dot(ca, sel_a) - _dot(cb, sel_b)
        acc = diff * diff if acc is None else acc + diff * diff
    return jnp.sqrt(acc + 1e-12)


def _a1(t1_ref, sampc_ref, f_ref, *, n, ns):
    samp_col = sampc_ref[0]
    iota_n = jax.lax.broadcasted_iota(jnp.int32, (ns, n), 1)
    S = _onehot(samp_col, iota_n)
    f_ref[0] = _dot(S, t1_ref[0])


def _a2(xyzs_ref, xyzst_ref, nd_ref, ni_ref, *, ns):
    xyz_s = xyzs_ref[0]
    xyz_sT = xyzst_ref[0]
    iota_s = jax.lax.broadcasted_iota(jnp.int32, (ns, ns), 1)
    d2 = None
    for c in range(3):
        diff = xyz_s[:, c:c + 1] - xyz_sT[c:c + 1, :]
        d2 = d2 + diff * diff if c else diff * diff
    dwork = jnp.sqrt(d2 + 1e-12)
    nd_cols, ni_cols = [], []
    for _ in range(K):
        m = jnp.min(dwork, axis=1, keepdims=True)
        cand = jnp.where(dwork <= m, iota_s, jnp.int32(ns))
        arg = jnp.min(cand, axis=1, keepdims=True)
        nd_cols.append(m)
        ni_cols.append(arg)
        dwork = jnp.where(iota_s == arg, _INF, dwork)
    nd_ref[0] = jnp.concatenate(nd_cols, axis=1)
    ni_ref[0] = jnp.concatenate(ni_cols, axis=1)


def _a3(ni_ref, xyzs_ref, sampc_ref, nd_ref, ac_ref, *, ns):
    f32 = jnp.float32
    ni = ni_ref[0]
    iota_s = jax.lax.broadcasted_iota(jnp.int32, (ns, ns), 1)
    t_r1 = jnp.concatenate(
        [xyzs_ref[0], sampc_ref[0].astype(f32), jnp.zeros((ns, 4), f32)],
        axis=1)
    ac_full = []
    for p in range(A):
        Gp = _onehot(ni[:, p:p + 1], iota_s)
        ac_full.append(_dot(Gp, t_r1))
    acx = jnp.concatenate([ac_full[p][:, 0:1] for p in range(A)], axis=1)
    acy = jnp.concatenate([ac_full[p][:, 1:2] for p in range(A)], axis=1)
    acz = jnp.concatenate([ac_full[p][:, 2:3] for p in range(A)], axis=1)
    samp_g = jnp.concatenate([ac_full[p][:, 3:4] for p in range(A)], axis=1)
    _, _, E1, E2 = _selectors(f32)
    subs = _sel_dists((acx, acy, acz), (acx, acy, acz), E1, E2)
    ia = jnp.concatenate([nd_ref[0][:, 1:A], subs], axis=1)
    ac_ref[0] = jnp.concatenate(
        [acx, acy, acz, samp_g, ia, jnp.zeros((ns, 2), f32)], axis=1)


def _a4(ni_ref, ac_ref, nd_ref, rel_ref, *, ns):
    f32 = jnp.float32
    ni = ni_ref[0]
    ac = ac_ref[0]
    acx, acy, acz = ac[:, 0:4], ac[:, 4:8], ac[:, 8:12]
    ia = ac[:, 16:22]
    iota_s = jax.lax.broadcasted_iota(jnp.int32, (ns, ns), 1)
    Psel, Qsel, E1, E2 = _selectors(f32)
    t_r2 = jnp.concatenate([acx, acy, acz, nd_ref[0][:, 1:A],
                            jnp.zeros((ns, 1), f32)], axis=1)
    rel_k = []
    for k in range(K):
        Gk = _onehot(ni[:, k:k + 1], iota_s)
        nac = _dot(Gk, t_r2)
        nacx, nacy, nacz = nac[:, 0:4], nac[:, 4:8], nac[:, 8:12]
        ndg = nac[:, 12:15]
        nsubs = _sel_dists((nacx, nacy, nacz), (nacx, nacy, nacz), E1, E2)
        neigh = jnp.concatenate([ndg, nsubs], axis=1)
        inter_k = _sel_dists((acx, acy, acz), (nacx, nacy, nacz), Psel, Qsel)
        rel_k.append(jnp.concatenate([ia, neigh, inter_k], axis=1))
    rel_ref[0] = jnp.concatenate(rel_k, axis=1)


def _a5(f_ref, ac_ref, t1_ref, adf_ref, *, n, ns):
    f32 = jnp.float32
    F = f_ref[0]
    ib_s = F[:, 67:73]
    nb_i = F[:, 73:77].astype(jnp.int32)
    ag_i = ac_ref[0][:, 12:16].astype(jnp.int32)
    ia = ac_ref[0][:, 16:22]
    xyzb = t1_ref[0][:, 0:3]
    iota_n = jax.lax.broadcasted_iota(jnp.int32, (ns, n), 1)
    cols = []
    for idx in (nb_i, ag_i):
        cx, cy, cz = [], [], []
        for p in range(A):
            r = _dot(_onehot(idx[:, p:p + 1], iota_n), xyzb)
            cx.append(r[:, 0:1]); cy.append(r[:, 1:2]); cz.append(r[:, 2:3])
        cols.append((jnp.concatenate(cx, axis=1),
                     jnp.concatenate(cy, axis=1),
                     jnp.concatenate(cz, axis=1)))
    Psel, Qsel, _, _ = _selectors(f32)
    inter_fin = _sel_dists(cols[0], cols[1], Psel, Qsel)
    adf_ref[0] = jnp.concatenate([ib_s, ia, inter_fin], axis=1)


def _b(fea_s_ref, adf_ref, Ww_ref, bw_ref, gw_ref, betw_ref, Wb_ref, bb_ref,
       gb_ref, betb_ref, Wo_ref, bo_ref, go_ref, beto_ref, out_ref):
    fea_s = fea_s_ref[...]
    adf = adf_ref[...]

    def bn(x, g, b):
        m = jnp.mean(x, axis=0, keepdims=True)
        v = jnp.mean((x - m) ** 2, axis=0, keepdims=True)
        return (x - m) / jnp.sqrt(v + 1e-5) * g + b

    def leaky(x):
        return jnp.where(x >= 0, x, 0.2 * x)

    w = bn(_dot(adf, Ww_ref[...]) + bw_ref[...], gw_ref[...], betw_ref[...])
    bi = bn(_dot(adf, Wb_ref[...]) + bb_ref[...], gb_ref[...], betb_ref[...])
    fea = leaky(fea_s * w + bi)
    cat = jnp.concatenate([fea, adf], axis=1)
    z = _dot(cat, Wo_ref[...]) + bo_ref[...]
    out_ref[...] = leaky(bn(z, go_ref[...], beto_ref[...]))


def _call(body, grid_b, in_arrays, in_blocks, out_blocks, out_shapes, **kw):
    return pl.pallas_call(
        functools.partial(body, **kw),
        grid=(grid_b,),
        in_specs=[pl.BlockSpec(s, lambda i, r=len(s) - 1: (i,) + (0,) * r)
                  for s in in_blocks],
        out_specs=[pl.BlockSpec(s, lambda i, r=len(s) - 1: (i,) + (0,) * r)
                   for s in out_blocks],
        out_shape=[jax.ShapeDtypeStruct(s, dt) for s, dt in out_shapes],
    )(*in_arrays)


def kernel(xyz, feature, raw_neighbors_feature, neighbors_idx_before,
           sample_indexes, Ww, bw, gw, betw, Wb, bb, gb, betb, Wo, bo, go,
           beto):
    b, n, _ = xyz.shape
    ns = sample_indexes.shape[1]
    f32 = jnp.float32

    intra_before = raw_neighbors_feature[:, :, 0, :INTRA]
    nb4 = neighbors_idx_before[:, :, :A].astype(f32)
    t1 = jnp.concatenate(
        [xyz, feature, intra_before, nb4, jnp.zeros((b, n, 3), f32)], axis=2)
    sampc = sample_indexes.reshape(b, ns, 1).astype(jnp.int32)

    (F,) = _call(_a1, b, (t1, sampc),
                 [(1, n, 80), (1, ns, 1)], [(1, ns, 80)],
                 [((b, ns, 80), f32)], n=n, ns=ns)
    xyz_s = F[:, :, 0:3]
    fea_s = F[:, :, 3:67]
    xyz_sT = jnp.swapaxes(xyz_s, 1, 2)

    nd, ni = _call(_a2, b, (xyz_s, xyz_sT),
                   [(1, ns, 3), (1, 3, ns)], [(1, ns, K), (1, ns, K)],
                   [((b, ns, K), f32), ((b, ns, K), jnp.int32)], ns=ns)

    (ac,) = _call(_a3, b, (ni, xyz_s, sampc, nd),
                  [(1, ns, K), (1, ns, 3), (1, ns, 1), (1, ns, K)],
                  [(1, ns, 24)], [((b, ns, 24), f32)], ns=ns)

    (rel,) = _call(_a4, b, (ni, ac, nd),
                   [(1, ns, K), (1, ns, 24), (1, ns, K)],
                   [(1, ns, K * CH)], [((b, ns, K * CH), f32)], ns=ns)

    (adf,) = _call(_a5, b, (F, ac, t1),
                   [(1, ns, 80), (1, ns, 24), (1, n, 80)],
                   [(1, ns, CH)], [((b, ns, CH), f32)], n=n, ns=ns)

    out = pl.pallas_call(
        _b, out_shape=jax.ShapeDtypeStruct((b * ns, 128), f32),
    )(fea_s.reshape(b * ns, 64), adf.reshape(b * ns, CH),
      Ww, bw.reshape(1, -1), gw.reshape(1, -1), betw.reshape(1, -1),
      Wb, bb.reshape(1, -1), gb.reshape(1, -1), betb.reshape(1, -1),
      Wo, bo.reshape(1, -1), go.reshape(1, -1), beto.reshape(1, -1))

    return (xyz_s, out.reshape(b, ns, 128), rel.reshape(b, ns, K, CH), ni)


# speedup vs baseline: 4.3773x; 2.2308x over previous
import functools
from itertools import combinations

import jax
import jax.numpy as jnp
from jax import lax
from jax.experimental import pallas as pl
from jax.experimental.pallas import tpu as pltpu
from jax.experimental.pallas import tpu_sc as plsc

A = 4
K = 16
INTRA = A * (A - 1) // 2
CH = 2 * INTRA + A * A
PAIRS = list(combinations(range(1, A), 2))

_INF = float("inf")
_HI = jax.lax.Precision.HIGHEST


def _dot(a, b, precision=_HI):
    return jnp.dot(a, b, preferred_element_type=jnp.float32,
                   precision=precision)


def _sc_gather(idx2d, table, per_b, tab_per_b):
    nrow, lw = idx2d.shape
    T = nrow * lw
    R, D = table.shape
    assert D == 128
    info = plsc.get_sparse_core_info()
    nw = info.num_cores * info.num_subcores
    nsub = nrow // nw
    mesh = plsc.VectorSubcoreMesh(core_axis_name="c", subcore_axis_name="s")

    @functools.partial(
        pl.kernel, mesh=mesh,
        out_type=jax.ShapeDtypeStruct((T, D), jnp.float32),
        scratch_types=[
            pltpu.VMEM((nsub, lw), jnp.int32),
            pltpu.VMEM((2, lw, D), jnp.float32),
            pltpu.SemaphoreType.DMA,
        ],
    )
    def k(idx_hbm, table_hbm, out_hbm, idx_v, rows_v, sem):
        wid = lax.axis_index("s") * info.num_cores + lax.axis_index("c")
        rbase = wid * nsub
        pltpu.sync_copy(idx_hbm.at[pl.ds(rbase, nsub)], idx_v)
        off = rbase * lw // per_b * tab_per_b
        for r in range(nsub):
            for c in range(lw // 16):
                sl = (r, pl.ds(c * 16, 16))
                idx_v[sl] = idx_v[sl] + off
        copies = []
        for r in range(nsub):
            copies.append(pltpu.async_copy(
                table_hbm.at[idx_v.at[r]], rows_v.at[r % 2], sem))
            if r >= 1:
                copies[r - 1].wait()
                pltpu.sync_copy(rows_v.at[(r - 1) % 2],
                                out_hbm.at[pl.ds((rbase + r - 1) * lw, lw)])
        copies[nsub - 1].wait()
        pltpu.sync_copy(rows_v.at[(nsub - 1) % 2],
                        out_hbm.at[pl.ds((rbase + nsub - 1) * lw, lw)])

    return k(idx2d, table)


def _selectors(f32):
    pq_col = jax.lax.broadcasted_iota(jnp.int32, (A, A * A), 1)
    pq_row = jax.lax.broadcasted_iota(jnp.int32, (A, A * A), 0)
    Psel = jnp.where(pq_col // A == pq_row, 1.0, 0.0).astype(f32)
    Qsel = jnp.where(pq_col % A == pq_row, 1.0, 0.0).astype(f32)
    pr_row = jax.lax.broadcasted_iota(jnp.int32, (A, len(PAIRS)), 0)
    pr_col = jax.lax.broadcasted_iota(jnp.int32, (A, len(PAIRS)), 1)
    E1 = jnp.zeros((A, len(PAIRS)), f32)
    E2 = jnp.zeros((A, len(PAIRS)), f32)
    for j, (p, q) in enumerate(PAIRS):
        hit = pr_col == j
        E1 = jnp.where(hit & (pr_row == p), 1.0, E1).astype(f32)
        E2 = jnp.where(hit & (pr_row == q), 1.0, E2).astype(f32)
    return Psel, Qsel, E1, E2


def _sel_dists(cols_a, cols_b, sel_a, sel_b):
    acc = None
    for ca, cb in zip(cols_a, cols_b):
        diff = _dot(ca, sel_a) - _dot(cb, sel_b)
        acc = diff * diff if acc is None else acc + diff * diff
    return jnp.sqrt(acc + 1e-12)


def _pick(arr, stride, offset):
    return jnp.concatenate(
        [arr[:, p * stride + offset:p * stride + offset + 1]
         for p in range(A)], axis=1)


def _a2(xyzs_ref, xyzst_ref, nd_ref, ni_ref, *, ns):
    xyz_s = xyzs_ref[0]
    xyz_sT = xyzst_ref[0]
    iota_s = jax.lax.broadcasted_iota(jnp.int32, (ns, ns), 1)
    d2 = None
    for c in range(3):
        diff = xyz_s[:, c:c + 1] - xyz_sT[c:c + 1, :]
        d2 = d2 + diff * diff if c else diff * diff
    dwork = jnp.sqrt(d2 + 1e-12)
    nd_cols, ni_cols = [], []
    for _ in range(K):
        m = jnp.min(dwork, axis=1, keepdims=True)
        cand = jnp.where(dwork <= m, iota_s, jnp.int32(ns))
        arg = jnp.min(cand, axis=1, keepdims=True)
        nd_cols.append(m)
        ni_cols.append(arg)
        dwork = jnp.where(iota_s == arg, _INF, dwork)
    nd_ref[0] = jnp.concatenate(nd_cols, axis=1)
    ni_ref[0] = jnp.concatenate(ni_cols, axis=1)


def _a3(ac2_ref, nd_ref, ac_ref, tr2_ref, agi_ref, *, ns):
    f32 = jnp.float32
    ac2 = ac2_ref[0]
    nd = nd_ref[0]
    acx = _pick(ac2, 128, 0)
    acy = _pick(ac2, 128, 1)
    acz = _pick(ac2, 128, 2)
    samp_g = _pick(ac2, 128, 3)
    _, _, E1, E2 = _selectors(f32)
    subs = _sel_dists((acx, acy, acz), (acx, acy, acz), E1, E2)
    ia = jnp.concatenate([nd[:, 1:A], subs], axis=1)
    ac_ref[0] = jnp.concatenate(
        [acx, acy, acz, samp_g, ia, jnp.zeros((ns, 2), f32)], axis=1)
    tr2_ref[0] = jnp.concatenate(
        [acx, acy, acz, nd[:, 1:A], jnp.zeros((ns, 113), f32)], axis=1)
    agi_ref[0] = samp_g.astype(jnp.int32)


def _a4(ac_ref, nac_ref, rel_ref, *, ns):
    f32 = jnp.float32
    ac = ac_ref[0]
    acx, acy, acz = ac[:, 0:4], ac[:, 4:8], ac[:, 8:12]
    ia = ac[:, 16:22]
    nac_all = nac_ref[0]
    Psel, Qsel, E1, E2 = _selectors(f32)
    rel_k = []
    for k in range(K):
        nac = nac_all[:, k * 128:k * 128 + 16]
        nacx, nacy, nacz = nac[:, 0:4], nac[:, 4:8], nac[:, 8:12]
        ndg = nac[:, 12:15]
        nsubs = _sel_dists((nacx, nacy, nacz), (nacx, nacy, nacz), E1, E2)
        neigh = jnp.concatenate([ndg, nsubs], axis=1)
        inter_k = _sel_dists((acx, acy, acz), (nacx, nacy, nacz), Psel, Qsel)
        rel_k.append(jnp.concatenate([ia, neigh, inter_k], axis=1))
    rel_ref[0] = jnp.concatenate(rel_k, axis=1)


def _a5(f_ref, ac_ref, xbnb_ref, xbag_ref, adf_ref, *, ns):
    f32 = jnp.float32
    ib_s = f_ref[0][:, 67:73]
    ia = ac_ref[0][:, 16:22]
    cols = []
    for ref in (xbnb_ref, xbag_ref):
        xb = ref[0]
        cols.append((_pick(xb, 128, 0), _pick(xb, 128, 1), _pick(xb, 128, 2)))
    Psel, Qsel, _, _ = _selectors(f32)
    inter_fin = _sel_dists(cols[0], cols[1], Psel, Qsel)
    adf_ref[0] = jnp.concatenate([ib_s, ia, inter_fin], axis=1)


def _b(fea_s_ref, adf_ref, Ww_ref, bw_ref, gw_ref, betw_ref, Wb_ref, bb_ref,
       gb_ref, betb_ref, Wo_ref, bo_ref, go_ref, beto_ref, out_ref):
    fea_s = fea_s_ref[...]
    adf = adf_ref[...]
    dflt = jax.lax.Precision.DEFAULT

    def bn(x, g, b):
        m = jnp.mean(x, axis=0, keepdims=True)
        v = jnp.mean((x - m) ** 2, axis=0, keepdims=True)
        return (x - m) / jnp.sqrt(v + 1e-5) * g + b

    def leaky(x):
        return jnp.where(x >= 0, x, 0.2 * x)

    w = bn(_dot(adf, Ww_ref[...], dflt) + bw_ref[...], gw_ref[...],
           betw_ref[...])
    bi = bn(_dot(adf, Wb_ref[...], dflt) + bb_ref[...], gb_ref[...],
            betb_ref[...])
    fea = leaky(fea_s * w + bi)
    cat = jnp.concatenate([fea, adf], axis=1)
    z = _dot(cat, Wo_ref[...], dflt) + bo_ref[...]
    out_ref[...] = leaky(bn(z, go_ref[...], beto_ref[...]))


def _call(body, grid_b, in_arrays, in_blocks, out_blocks, out_shapes, **kw):
    return pl.pallas_call(
        functools.partial(body, **kw),
        grid=(grid_b,),
        in_specs=[pl.BlockSpec(s, lambda i, r=len(s) - 1: (i,) + (0,) * r)
                  for s in in_blocks],
        out_specs=[pl.BlockSpec(s, lambda i, r=len(s) - 1: (i,) + (0,) * r)
                   for s in out_blocks],
        out_shape=[jax.ShapeDtypeStruct(s, dt) for s, dt in out_shapes],
    )(*in_arrays)


def kernel(xyz, feature, raw_neighbors_feature, neighbors_idx_before,
           sample_indexes, Ww, bw, gw, betw, Wb, bb, gb, betb, Wo, bo, go,
           beto):
    b, n, _ = xyz.shape
    ns = sample_indexes.shape[1]
    f32 = jnp.float32
    i32 = jnp.int32

    intra_before = raw_neighbors_feature[:, :, 0, :INTRA]
    nb4f = neighbors_idx_before[:, :, :A].astype(f32)
    t1 = jnp.concatenate(
        [xyz, feature, intra_before, nb4f,
         jnp.zeros((b, n, 51), f32)], axis=2)
    samp = sample_indexes.astype(i32)

    F = _sc_gather(samp.reshape(-1, 128), t1.reshape(b * n, 128),
                   per_b=ns, tab_per_b=n).reshape(b, ns, 128)
    xyz_s = F[:, :, 0:3]
    fea_s = F[:, :, 3:67]
    xyz_sT = jnp.swapaxes(xyz_s, 1, 2)

    nd, ni = _call(_a2, b, (xyz_s, xyz_sT),
                   [(1, ns, 3), (1, 3, ns)], [(1, ns, K), (1, ns, K)],
                   [((b, ns, K), f32), ((b, ns, K), i32)], ns=ns)

    t_r1 = jnp.concatenate(
        [xyz_s, samp.reshape(b, ns, 1).astype(f32),
         jnp.zeros((b, ns, 124), f32)], axis=2)
    ac2 = _sc_gather(ni[:, :, :A].reshape(-1, 128),
                     t_r1.reshape(b * ns, 128),
                     per_b=ns * A, tab_per_b=ns).reshape(b, ns, A * 128)

    ac, t_r2, ag_i = _call(
        _a3, b, (ac2, nd),
        [(1, ns, A * 128), (1, ns, K)],
        [(1, ns, 24), (1, ns, 128), (1, ns, A)],
        [((b, ns, 24), f32), ((b, ns, 128), f32), ((b, ns, A), i32)], ns=ns)

    nac = _sc_gather(ni.reshape(-1, 128), t_r2.reshape(b * ns, 128),
                     per_b=ns * K, tab_per_b=ns).reshape(b, ns, K * 128)

    (rel,) = _call(_a4, b, (ac, nac),
                   [(1, ns, 24), (1, ns, K * 128)],
                   [(1, ns, K * CH)], [((b, ns, K * CH), f32)], ns=ns)

    xyzpad = jnp.concatenate([xyz, jnp.zeros((b, n, 125), f32)], axis=2)
    nb_i = F[:, :, 73:77].astype(i32)
    idx_ba = jnp.concatenate(
        [nb_i.reshape(b, ns * A), ag_i.reshape(b, ns * A)], axis=1)
    xb = _sc_gather(idx_ba.reshape(-1, 128), xyzpad.reshape(b * n, 128),
                    per_b=2 * ns * A, tab_per_b=n).reshape(b, 2, ns, A * 128)

    (adf,) = _call(_a5, b, (F, ac, xb[:, 0], xb[:, 1]),
                   [(1, ns, 128), (1, ns, 24), (1, ns, A * 128),
                    (1, ns, A * 128)],
                   [(1, ns, CH)], [((b, ns, CH), f32)], ns=ns)

    out = pl.pallas_call(
        _b, out_shape=jax.ShapeDtypeStruct((b * ns, 128), f32),
    )(fea_s.reshape(b * ns, 64), adf.reshape(b * ns, CH),
      Ww, bw.reshape(1, -1), gw.reshape(1, -1), betw.reshape(1, -1),
      Wb, bb.reshape(1, -1), gb.reshape(1, -1), betb.reshape(1, -1),
      Wo, bo.reshape(1, -1), go.reshape(1, -1), beto.reshape(1, -1))

    return (xyz_s, out.reshape(b, ns, 128), rel.reshape(b, ns, K, CH), ni)


# speedup vs baseline: 7.3404x; 1.6769x over previous
import functools
from itertools import combinations

import jax
import jax.numpy as jnp
from jax import lax
from jax.experimental import pallas as pl
from jax.experimental.pallas import tpu as pltpu
from jax.experimental.pallas import tpu_sc as plsc

A = 4
K = 16
INTRA = A * (A - 1) // 2
CH = 2 * INTRA + A * A
PAIRS = list(combinations(range(1, A), 2))

_INF = float("inf")
_HI = jax.lax.Precision.HIGHEST


def _dot(a, b, precision=_HI):
    return jnp.dot(a, b, preferred_element_type=jnp.float32,
                   precision=precision)


def _sc_gather(idx2d, table, per_b, tab_per_b):
    nrow, lw = idx2d.shape
    T = nrow * lw
    R, D = table.shape
    assert D == 128
    info = plsc.get_sparse_core_info()
    nw = info.num_cores * info.num_subcores
    nsub = nrow // nw
    mesh = plsc.VectorSubcoreMesh(core_axis_name="c", subcore_axis_name="s")

    @functools.partial(
        pl.kernel, mesh=mesh,
        out_type=jax.ShapeDtypeStruct((T, D), jnp.float32),
        scratch_types=[
            pltpu.VMEM((nsub, lw), jnp.int32),
            pltpu.VMEM((2, lw, D), jnp.float32),
            pltpu.SemaphoreType.DMA,
        ],
    )
    def k(idx_hbm, table_hbm, out_hbm, idx_v, rows_v, sem):
        wid = lax.axis_index("s") * info.num_cores + lax.axis_index("c")
        rbase = wid * nsub
        pltpu.sync_copy(idx_hbm.at[pl.ds(rbase, nsub)], idx_v)
        off = rbase * lw // per_b * tab_per_b
        for r in range(nsub):
            for c in range(lw // 16):
                sl = (r, pl.ds(c * 16, 16))
                idx_v[sl] = idx_v[sl] + off
        copies = []
        for r in range(nsub):
            copies.append(pltpu.async_copy(
                table_hbm.at[idx_v.at[r]], rows_v.at[r % 2], sem))
            if r >= 1:
                copies[r - 1].wait()
                pltpu.sync_copy(rows_v.at[(r - 1) % 2],
                                out_hbm.at[pl.ds((rbase + r - 1) * lw, lw)])
        copies[nsub - 1].wait()
        pltpu.sync_copy(rows_v.at[(nsub - 1) % 2],
                        out_hbm.at[pl.ds((rbase + nsub - 1) * lw, lw)])

    return k(idx2d, table)


def _selectors(f32):
    pq_col = jax.lax.broadcasted_iota(jnp.int32, (A, A * A), 1)
    pq_row = jax.lax.broadcasted_iota(jnp.int32, (A, A * A), 0)
    Psel = jnp.where(pq_col // A == pq_row, 1.0, 0.0).astype(f32)
    Qsel = jnp.where(pq_col % A == pq_row, 1.0, 0.0).astype(f32)
    pr_row = jax.lax.broadcasted_iota(jnp.int32, (A, len(PAIRS)), 0)
    pr_col = jax.lax.broadcasted_iota(jnp.int32, (A, len(PAIRS)), 1)
    E1 = jnp.zeros((A, len(PAIRS)), f32)
    E2 = jnp.zeros((A, len(PAIRS)), f32)
    for j, (p, q) in enumerate(PAIRS):
        hit = pr_col == j
        E1 = jnp.where(hit & (pr_row == p), 1.0, E1).astype(f32)
        E2 = jnp.where(hit & (pr_row == q), 1.0, E2).astype(f32)
    return Psel, Qsel, E1, E2


def _sel_dists(cols_a, cols_b, sel_a, sel_b):
    acc = None
    for ca, cb in zip(cols_a, cols_b):
        diff = _dot(ca, sel_a) - _dot(cb, sel_b)
        acc = diff * diff if acc is None else acc + diff * diff
    return jnp.sqrt(acc + 1e-12)


def _pick(arr, stride, offset):
    return jnp.concatenate(
        [arr[:, p * stride + offset:p * stride + offset + 1]
         for p in range(A)], axis=1)


def _a2(xyzs_ref, xyzst_ref, nd_ref, ni_ref, *, ns):
    xyz_s = xyzs_ref[0]
    xyz_sT = xyzst_ref[0]
    iota_s = jax.lax.broadcasted_iota(jnp.int32, (ns, ns), 1)
    d2 = None
    for c in range(3):
        diff = xyz_s[:, c:c + 1] - xyz_sT[c:c + 1, :]
        d2 = d2 + diff * diff if c else diff * diff
    dwork = jnp.sqrt(d2 + 1e-12)
    nd_cols, ni_cols = [], []
    for _ in range(K):
        m = jnp.min(dwork, axis=1, keepdims=True)
        cand = jnp.where(dwork <= m, iota_s, jnp.int32(ns))
        arg = jnp.min(cand, axis=1, keepdims=True)
        nd_cols.append(m)
        ni_cols.append(arg)
        dwork = jnp.where(iota_s == arg, _INF, dwork)
    nd_ref[0] = jnp.concatenate(nd_cols, axis=1)
    ni_ref[0] = jnp.concatenate(ni_cols, axis=1)


def _a3(ac2_ref, nd_ref, ac_ref, tr2_ref, agi_ref, *, ns):
    f32 = jnp.float32
    ac2 = ac2_ref[0]
    nd = nd_ref[0]
    acx = _pick(ac2, 128, 0)
    acy = _pick(ac2, 128, 1)
    acz = _pick(ac2, 128, 2)
    samp_g = _pick(ac2, 128, 3)
    _, _, E1, E2 = _selectors(f32)
    subs = _sel_dists((acx, acy, acz), (acx, acy, acz), E1, E2)
    ia = jnp.concatenate([nd[:, 1:A], subs], axis=1)
    ac_ref[0] = jnp.concatenate(
        [acx, acy, acz, samp_g, ia, jnp.zeros((ns, 2), f32)], axis=1)
    tr2_ref[0] = jnp.concatenate(
        [acx, acx, acx, acx, acy, acy, acy, acy, acz, acz, acz, acz,
         nd[:, 1:A], subs, jnp.zeros((ns, 74), f32)], axis=1)
    agi_ref[0] = samp_g.astype(jnp.int32)


def _a4(ac_ref, nac_ref, rel_ref, *, ns):
    f32 = jnp.float32
    ac = ac_ref[0]
    acx, acy, acz = ac[:, 0:4], ac[:, 4:8], ac[:, 8:12]
    ia = ac[:, 16:22]
    nac_all = nac_ref[0]
    m_col = jax.lax.broadcasted_iota(jnp.int32, (A, A * A), 1)
    m_row = jax.lax.broadcasted_iota(jnp.int32, (A, A * A), 0)
    RepP = jnp.where(m_col // A == m_row, 1.0, 0.0).astype(f32)
    acp48 = jnp.concatenate(
        [_dot(acx, RepP), _dot(acy, RepP), _dot(acz, RepP)], axis=1)
    rel_k = []
    for k in range(K):
        blk = nac_all[:, k * 128:k * 128 + 48]
        diff = acp48 - blk
        sq = diff * diff
        d2 = sq[:, 0:16] + sq[:, 16:32] + sq[:, 32:48]
        inter_k = jnp.sqrt(d2 + 1e-12)
        neigh = nac_all[:, k * 128 + 48:k * 128 + 54]
        rel_k.append(jnp.concatenate([ia, neigh, inter_k], axis=1))
    rel_ref[0] = jnp.concatenate(rel_k, axis=1)


def _a5(f_ref, ac_ref, xbnb_ref, xbag_ref, adf_ref, *, ns):
    f32 = jnp.float32
    ib_s = f_ref[0][:, 67:73]
    ia = ac_ref[0][:, 16:22]
    cols = []
    for ref in (xbnb_ref, xbag_ref):
        xb = ref[0]
        cols.append((_pick(xb, 128, 0), _pick(xb, 128, 1), _pick(xb, 128, 2)))
    Psel, Qsel, _, _ = _selectors(f32)
    inter_fin = _sel_dists(cols[0], cols[1], Psel, Qsel)
    adf_ref[0] = jnp.concatenate([ib_s, ia, inter_fin], axis=1)


def _b(fea_s_ref, adf_ref, Ww_ref, bw_ref, gw_ref, betw_ref, Wb_ref, bb_ref,
       gb_ref, betb_ref, Wo_ref, bo_ref, go_ref, beto_ref, out_ref):
    fea_s = fea_s_ref[...]
    adf = adf_ref[...]
    dflt = jax.lax.Precision.DEFAULT

    def bn(x, g, b):
        m = jnp.mean(x, axis=0, keepdims=True)
        v = jnp.mean((x - m) ** 2, axis=0, keepdims=True)
        return (x - m) / jnp.sqrt(v + 1e-5) * g + b

    def leaky(x):
        return jnp.where(x >= 0, x, 0.2 * x)

    w = bn(_dot(adf, Ww_ref[...], dflt) + bw_ref[...], gw_ref[...],
           betw_ref[...])
    bi = bn(_dot(adf, Wb_ref[...], dflt) + bb_ref[...], gb_ref[...],
            betb_ref[...])
    fea = leaky(fea_s * w + bi)
    cat = jnp.concatenate([fea, adf], axis=1)
    z = _dot(cat, Wo_ref[...], dflt) + bo_ref[...]
    out_ref[...] = leaky(bn(z, go_ref[...], beto_ref[...]))


def _call(body, grid_b, in_arrays, in_blocks, out_blocks, out_shapes, **kw):
    return pl.pallas_call(
        functools.partial(body, **kw),
        grid=(grid_b,),
        in_specs=[pl.BlockSpec(s, lambda i, r=len(s) - 1: (i,) + (0,) * r)
                  for s in in_blocks],
        out_specs=[pl.BlockSpec(s, lambda i, r=len(s) - 1: (i,) + (0,) * r)
                   for s in out_blocks],
        out_shape=[jax.ShapeDtypeStruct(s, dt) for s, dt in out_shapes],
    )(*in_arrays)


def kernel(xyz, feature, raw_neighbors_feature, neighbors_idx_before,
           sample_indexes, Ww, bw, gw, betw, Wb, bb, gb, betb, Wo, bo, go,
           beto):
    b, n, _ = xyz.shape
    ns = sample_indexes.shape[1]
    f32 = jnp.float32
    i32 = jnp.int32

    intra_before = raw_neighbors_feature[:, :, 0, :INTRA]
    nb4f = neighbors_idx_before[:, :, :A].astype(f32)
    t1 = jnp.concatenate(
        [xyz, feature, intra_before, nb4f,
         jnp.zeros((b, n, 51), f32)], axis=2)
    samp = sample_indexes.astype(i32)

    F = _sc_gather(samp.reshape(-1, 128), t1.reshape(b * n, 128),
                   per_b=ns, tab_per_b=n).reshape(b, ns, 128)
    xyz_s = F[:, :, 0:3]
    fea_s = F[:, :, 3:67]
    xyz_sT = jnp.swapaxes(xyz_s, 1, 2)

    nd, ni = _call(_a2, b, (xyz_s, xyz_sT),
                   [(1, ns, 3), (1, 3, ns)], [(1, ns, K), (1, ns, K)],
                   [((b, ns, K), f32), ((b, ns, K), i32)], ns=ns)

    t_r1 = jnp.concatenate(
        [xyz_s, samp.reshape(b, ns, 1).astype(f32),
         jnp.zeros((b, ns, 124), f32)], axis=2)
    ac2 = _sc_gather(ni[:, :, :A].reshape(-1, 128),
                     t_r1.reshape(b * ns, 128),
                     per_b=ns * A, tab_per_b=ns).reshape(b, ns, A * 128)

    ac, t_r2, ag_i = _call(
        _a3, b, (ac2, nd),
        [(1, ns, A * 128), (1, ns, K)],
        [(1, ns, 24), (1, ns, 128), (1, ns, A)],
        [((b, ns, 24), f32), ((b, ns, 128), f32), ((b, ns, A), i32)], ns=ns)

    nac = _sc_gather(ni.reshape(-1, 128), t_r2.reshape(b * ns, 128),
                     per_b=ns * K, tab_per_b=ns).reshape(b, ns, K * 128)

    (rel,) = _call(_a4, b, (ac, nac),
                   [(1, ns, 24), (1, ns, K * 128)],
                   [(1, ns, K * CH)], [((b, ns, K * CH), f32)], ns=ns)

    xyzpad = jnp.concatenate([xyz, jnp.zeros((b, n, 125), f32)], axis=2)
    nb_i = F[:, :, 73:77].astype(i32)
    idx_ba = jnp.concatenate(
        [nb_i.reshape(b, ns * A), ag_i.reshape(b, ns * A)], axis=1)
    xb = _sc_gather(idx_ba.reshape(-1, 128), xyzpad.reshape(b * n, 128),
                    per_b=2 * ns * A, tab_per_b=n).reshape(b, 2, ns, A * 128)

    (adf,) = _call(_a5, b, (F, ac, xb[:, 0], xb[:, 1]),
                   [(1, ns, 128), (1, ns, 24), (1, ns, A * 128),
                    (1, ns, A * 128)],
                   [(1, ns, CH)], [((b, ns, CH), f32)], ns=ns)

    out = pl.pallas_call(
        _b, out_shape=jax.ShapeDtypeStruct((b * ns, 128), f32),
    )(fea_s.reshape(b * ns, 64), adf.reshape(b * ns, CH),
      Ww, bw.reshape(1, -1), gw.reshape(1, -1), betw.reshape(1, -1),
      Wb, bb.reshape(1, -1), gb.reshape(1, -1), betb.reshape(1, -1),
      Wo, bo.reshape(1, -1), go.reshape(1, -1), beto.reshape(1, -1))

    return (xyz_s, out.reshape(b, ns, 128), rel.reshape(b, ns, K, CH), ni)


# speedup vs baseline: 7.5297x; 1.0258x over previous
import functools
from itertools import combinations

import jax
import jax.numpy as jnp
from jax import lax
from jax.experimental import pallas as pl
from jax.experimental.pallas import tpu as pltpu
from jax.experimental.pallas import tpu_sc as plsc

A = 4
K = 16
INTRA = A * (A - 1) // 2
CH = 2 * INTRA + A * A
PAIRS = list(combinations(range(1, A), 2))

_INF = float("inf")
_HI = jax.lax.Precision.HIGHEST


def _dot(a, b, precision=_HI):
    return jnp.dot(a, b, preferred_element_type=jnp.float32,
                   precision=precision)


def _sc_gather(idx2d, table, per_b, tab_per_b):
    nrow, lw = idx2d.shape
    T = nrow * lw
    R, D = table.shape
    assert D == 128
    info = plsc.get_sparse_core_info()
    nw = info.num_cores * info.num_subcores
    nsub = nrow // nw
    mesh = plsc.VectorSubcoreMesh(core_axis_name="c", subcore_axis_name="s")

    @functools.partial(
        pl.kernel, mesh=mesh,
        out_type=jax.ShapeDtypeStruct((T, D), jnp.float32),
        scratch_types=[
            pltpu.VMEM((nsub, lw), jnp.int32),
            pltpu.VMEM((2, lw, D), jnp.float32),
            pltpu.SemaphoreType.DMA,
        ],
    )
    def k(idx_hbm, table_hbm, out_hbm, idx_v, rows_v, sem):
        wid = lax.axis_index("s") * info.num_cores + lax.axis_index("c")
        rbase = wid * nsub
        pltpu.sync_copy(idx_hbm.at[pl.ds(rbase, nsub)], idx_v)
        off = rbase * lw // per_b * tab_per_b
        for r in range(nsub):
            for c in range(lw // 16):
                sl = (r, pl.ds(c * 16, 16))
                idx_v[sl] = idx_v[sl] + off
        copies = []
        for r in range(nsub):
            copies.append(pltpu.async_copy(
                table_hbm.at[idx_v.at[r]], rows_v.at[r % 2], sem))
            if r >= 1:
                copies[r - 1].wait()
                pltpu.sync_copy(rows_v.at[(r - 1) % 2],
                                out_hbm.at[pl.ds((rbase + r - 1) * lw, lw)])
        copies[nsub - 1].wait()
        pltpu.sync_copy(rows_v.at[(nsub - 1) % 2],
                        out_hbm.at[pl.ds((rbase + nsub - 1) * lw, lw)])

    return k(idx2d, table)


def _selectors(f32):
    pq_col = jax.lax.broadcasted_iota(jnp.int32, (A, A * A), 1)
    pq_row = jax.lax.broadcasted_iota(jnp.int32, (A, A * A), 0)
    Psel = jnp.where(pq_col // A == pq_row, 1.0, 0.0).astype(f32)
    Qsel = jnp.where(pq_col % A == pq_row, 1.0, 0.0).astype(f32)
    pr_row = jax.lax.broadcasted_iota(jnp.int32, (A, len(PAIRS)), 0)
    pr_col = jax.lax.broadcasted_iota(jnp.int32, (A, len(PAIRS)), 1)
    E1 = jnp.zeros((A, len(PAIRS)), f32)
    E2 = jnp.zeros((A, len(PAIRS)), f32)
    for j, (p, q) in enumerate(PAIRS):
        hit = pr_col == j
        E1 = jnp.where(hit & (pr_row == p), 1.0, E1).astype(f32)
        E2 = jnp.where(hit & (pr_row == q), 1.0, E2).astype(f32)
    return Psel, Qsel, E1, E2


def _sel_dists(cols_a, cols_b, sel_a, sel_b):
    acc = None
    for ca, cb in zip(cols_a, cols_b):
        diff = _dot(ca, sel_a) - _dot(cb, sel_b)
        acc = diff * diff if acc is None else acc + diff * diff
    return jnp.sqrt(acc + 1e-12)


def _pick(arr, stride, offset):
    return jnp.concatenate(
        [arr[:, p * stride + offset:p * stride + offset + 1]
         for p in range(A)], axis=1)


def _a2(xyzs_ref, xyzst_ref, nd_ref, ni_ref, *, ns):
    xyz_s = xyzs_ref[0]
    xyz_sT = xyzst_ref[0]
    iota_s = jax.lax.broadcasted_iota(jnp.int32, (ns, ns), 1)
    d2 = None
    for c in range(3):
        diff = xyz_s[:, c:c + 1] - xyz_sT[c:c + 1, :]
        d2 = d2 + diff * diff if c else diff * diff
    dwork = jnp.sqrt(d2 + 1e-12)
    nd_cols, ni_cols = [], []
    for _ in range(K):
        m = jnp.min(dwork, axis=1, keepdims=True)
        cand = jnp.where(dwork <= m, iota_s, jnp.int32(ns))
        arg = jnp.min(cand, axis=1, keepdims=True)
        nd_cols.append(m)
        ni_cols.append(arg)
        dwork = jnp.where(iota_s == arg, _INF, dwork)
    nd_ref[0] = jnp.concatenate(nd_cols, axis=1)
    ni_ref[0] = jnp.concatenate(ni_cols, axis=1)


def _a3(ac2_ref, nd_ref, ac_ref, tr2_ref, agi_ref, *, ns):
    f32 = jnp.float32
    ac2 = ac2_ref[0]
    nd = nd_ref[0]
    acx = _pick(ac2, 128, 0)
    acy = _pick(ac2, 128, 1)
    acz = _pick(ac2, 128, 2)
    samp_g = _pick(ac2, 128, 77)
    _, _, E1, E2 = _selectors(f32)
    subs = _sel_dists((acx, acy, acz), (acx, acy, acz), E1, E2)
    ia = jnp.concatenate([nd[:, 1:A], subs], axis=1)
    ac_ref[0] = jnp.concatenate(
        [acx, acy, acz, samp_g, ia, jnp.zeros((ns, 2), f32)], axis=1)
    tr2_ref[0] = jnp.concatenate(
        [acx, acx, acx, acx, acy, acy, acy, acy, acz, acz, acz, acz,
         nd[:, 1:A], subs, jnp.zeros((ns, 74), f32)], axis=1)
    agi_ref[0] = samp_g.astype(jnp.int32)


def _a45(ac_ref, nac_ref, f_ref, xbnb_ref, xbag_ref, rel_ref, adf_ref, *, ns):
    f32 = jnp.float32
    ac = ac_ref[0]
    acx, acy, acz = ac[:, 0:4], ac[:, 4:8], ac[:, 8:12]
    ia = ac[:, 16:22]
    nac_all = nac_ref[0]
    m_col = jax.lax.broadcasted_iota(jnp.int32, (A, A * A), 1)
    m_row = jax.lax.broadcasted_iota(jnp.int32, (A, A * A), 0)
    RepP = jnp.where(m_col // A == m_row, 1.0, 0.0).astype(f32)
    acp48 = jnp.concatenate(
        [_dot(acx, RepP), _dot(acy, RepP), _dot(acz, RepP)], axis=1)
    rel_k = []
    for k in range(K):
        blk = nac_all[:, k * 128:k * 128 + 48]
        diff = acp48 - blk
        sq = diff * diff
        d2 = sq[:, 0:16] + sq[:, 16:32] + sq[:, 32:48]
        inter_k = jnp.sqrt(d2 + 1e-12)
        neigh = nac_all[:, k * 128 + 48:k * 128 + 54]
        rel_k.append(jnp.concatenate([ia, neigh, inter_k], axis=1))
    rel_ref[0] = jnp.concatenate(rel_k, axis=1)
    ib_s = f_ref[0][:, 67:73]
    cols = []
    for ref in (xbnb_ref, xbag_ref):
        xb = ref[0][0]
        cols.append((_pick(xb, 128, 0), _pick(xb, 128, 1), _pick(xb, 128, 2)))
    Psel, Qsel, _, _ = _selectors(f32)
    inter_fin = _sel_dists(cols[0], cols[1], Psel, Qsel)
    adf_ref[0] = jnp.concatenate([ib_s, ia, inter_fin], axis=1)


def _b(fea_s_ref, adf_ref, Ww_ref, bw_ref, gw_ref, betw_ref, Wb_ref, bb_ref,
       gb_ref, betb_ref, Wo_ref, bo_ref, go_ref, beto_ref, out_ref):
    fea_s = fea_s_ref[...]
    adf = adf_ref[...]
    dflt = jax.lax.Precision.DEFAULT

    def bn(x, g, b):
        m = jnp.mean(x, axis=0, keepdims=True)
        v = jnp.mean((x - m) ** 2, axis=0, keepdims=True)
        return (x - m) / jnp.sqrt(v + 1e-5) * g + b

    def leaky(x):
        return jnp.where(x >= 0, x, 0.2 * x)

    w = bn(_dot(adf, Ww_ref[...], dflt) + bw_ref[...], gw_ref[...],
           betw_ref[...])
    bi = bn(_dot(adf, Wb_ref[...], dflt) + bb_ref[...], gb_ref[...],
            betb_ref[...])
    fea = leaky(fea_s * w + bi)
    cat = jnp.concatenate([fea, adf], axis=1)
    z = _dot(cat, Wo_ref[...], dflt) + bo_ref[...]
    out_ref[...] = leaky(bn(z, go_ref[...], beto_ref[...]))


def _call(body, grid_b, in_arrays, in_blocks, out_blocks, out_shapes, **kw):
    return pl.pallas_call(
        functools.partial(body, **kw),
        grid=(grid_b,),
        in_specs=[pl.BlockSpec(s, lambda i, r=len(s) - 1: (i,) + (0,) * r)
                  for s in in_blocks],
        out_specs=[pl.BlockSpec(s, lambda i, r=len(s) - 1: (i,) + (0,) * r)
                   for s in out_blocks],
        out_shape=[jax.ShapeDtypeStruct(s, dt) for s, dt in out_shapes],
    )(*in_arrays)


def kernel(xyz, feature, raw_neighbors_feature, neighbors_idx_before,
           sample_indexes, Ww, bw, gw, betw, Wb, bb, gb, betb, Wo, bo, go,
           beto):
    b, n, _ = xyz.shape
    ns = sample_indexes.shape[1]
    f32 = jnp.float32
    i32 = jnp.int32

    intra_before = raw_neighbors_feature[:, :, 0, :INTRA]
    nb4f = neighbors_idx_before[:, :, :A].astype(f32)
    row_id = jax.lax.broadcasted_iota(f32, (b, n, 1), 1)
    t1 = jnp.concatenate(
        [xyz, feature, intra_before, nb4f, row_id,
         jnp.zeros((b, n, 50), f32)], axis=2)
    samp = sample_indexes.astype(i32)

    F = _sc_gather(samp.reshape(-1, 128), t1.reshape(b * n, 128),
                   per_b=ns, tab_per_b=n).reshape(b, ns, 128)
    xyz_s = F[:, :, 0:3]
    fea_s = F[:, :, 3:67]
    xyz_sT = jnp.swapaxes(xyz_s, 1, 2)

    nd, ni = _call(_a2, b, (xyz_s, xyz_sT),
                   [(1, ns, 3), (1, 3, ns)], [(1, ns, K), (1, ns, K)],
                   [((b, ns, K), f32), ((b, ns, K), i32)], ns=ns)

    ac2 = _sc_gather(ni[:, :, :A].reshape(-1, 128),
                     F.reshape(b * ns, 128),
                     per_b=ns * A, tab_per_b=ns).reshape(b, ns, A * 128)

    ac, t_r2, ag_i = _call(
        _a3, b, (ac2, nd),
        [(1, ns, A * 128), (1, ns, K)],
        [(1, ns, 24), (1, ns, 128), (1, ns, A)],
        [((b, ns, 24), f32), ((b, ns, 128), f32), ((b, ns, A), i32)], ns=ns)

    nac = _sc_gather(ni.reshape(-1, 128), t_r2.reshape(b * ns, 128),
                     per_b=ns * K, tab_per_b=ns).reshape(b, ns, K * 128)

    nb_i = F[:, :, 73:77].astype(i32)
    idx_ba = jnp.concatenate(
        [nb_i.reshape(b, ns * A), ag_i.reshape(b, ns * A)], axis=1)
    xb = _sc_gather(idx_ba.reshape(-1, 128), t1.reshape(b * n, 128),
                    per_b=2 * ns * A, tab_per_b=n).reshape(b, 2, ns, A * 128)

    rel, adf = pl.pallas_call(
        functools.partial(_a45, ns=ns),
        grid=(b,),
        in_specs=[
            pl.BlockSpec((1, ns, 24), lambda i: (i, 0, 0)),
            pl.BlockSpec((1, ns, K * 128), lambda i: (i, 0, 0)),
            pl.BlockSpec((1, ns, 128), lambda i: (i, 0, 0)),
            pl.BlockSpec((1, 1, ns, A * 128), lambda i: (i, 0, 0, 0)),
            pl.BlockSpec((1, 1, ns, A * 128), lambda i: (i, 1, 0, 0)),
        ],
        out_specs=[
            pl.BlockSpec((1, ns, K * CH), lambda i: (i, 0, 0)),
            pl.BlockSpec((1, ns, CH), lambda i: (i, 0, 0)),
        ],
        out_shape=[jax.ShapeDtypeStruct((b, ns, K * CH), f32),
                   jax.ShapeDtypeStruct((b, ns, CH), f32)],
    )(ac, nac, F, xb, xb)

    out = pl.pallas_call(
        _b, out_shape=jax.ShapeDtypeStruct((b * ns, 128), f32),
    )(fea_s.reshape(b * ns, 64), adf.reshape(b * ns, CH),
      Ww, bw.reshape(1, -1), gw.reshape(1, -1), betw.reshape(1, -1),
      Wb, bb.reshape(1, -1), gb.reshape(1, -1), betb.reshape(1, -1),
      Wo, bo.reshape(1, -1), go.reshape(1, -1), beto.reshape(1, -1))

    return (xyz_s, out.reshape(b, ns, 128), rel.reshape(b, ns, K, CH), ni)


# speedup vs baseline: 7.6296x; 1.0133x over previous
import functools
from itertools import combinations

import jax
import jax.numpy as jnp
from jax import lax
from jax.experimental import pallas as pl
from jax.experimental.pallas import tpu as pltpu
from jax.experimental.pallas import tpu_sc as plsc

A = 4
K = 16
INTRA = A * (A - 1) // 2
CH = 2 * INTRA + A * A
PAIRS = list(combinations(range(1, A), 2))

_INF = float("inf")
_HI = jax.lax.Precision.HIGHEST


def _dot(a, b, precision=_HI):
    return jnp.dot(a, b, preferred_element_type=jnp.float32,
                   precision=precision)


def _sc_gather(idx2d, table, per_b, tab_per_b):
    nrow, lw = idx2d.shape
    T = nrow * lw
    R, D = table.shape
    assert D == 128
    info = plsc.get_sparse_core_info()
    nw = info.num_cores * info.num_subcores
    nsub = nrow // nw
    mesh = plsc.VectorSubcoreMesh(core_axis_name="c", subcore_axis_name="s")

    @functools.partial(
        pl.kernel, mesh=mesh,
        out_type=jax.ShapeDtypeStruct((T, D), jnp.float32),
        scratch_types=[
            pltpu.VMEM((nsub, lw), jnp.int32),
            pltpu.VMEM((2, lw, D), jnp.float32),
            pltpu.SemaphoreType.DMA,
        ],
    )
    def k(idx_hbm, table_hbm, out_hbm, idx_v, rows_v, sem):
        wid = lax.axis_index("s") * info.num_cores + lax.axis_index("c")
        rbase = wid * nsub
        pltpu.sync_copy(idx_hbm.at[pl.ds(rbase, nsub)], idx_v)
        off = rbase * lw // per_b * tab_per_b
        for r in range(nsub):
            for c in range(lw // 16):
                sl = (r, pl.ds(c * 16, 16))
                idx_v[sl] = idx_v[sl] + off
        copies = []
        for r in range(nsub):
            copies.append(pltpu.async_copy(
                table_hbm.at[idx_v.at[r]], rows_v.at[r % 2], sem))
            if r >= 1:
                copies[r - 1].wait()
                pltpu.sync_copy(rows_v.at[(r - 1) % 2],
                                out_hbm.at[pl.ds((rbase + r - 1) * lw, lw)])
        copies[nsub - 1].wait()
        pltpu.sync_copy(rows_v.at[(nsub - 1) % 2],
                        out_hbm.at[pl.ds((rbase + nsub - 1) * lw, lw)])

    return k(idx2d, table)


def _sc_gather2(idx2d_a, table_a, per_b_a, tab_per_b_a,
                idx2d_b, table_b, per_b_b, tab_per_b_b):
    nrow_a, lw = idx2d_a.shape
    nrow_b, _ = idx2d_b.shape
    D = table_a.shape[1]
    info = plsc.get_sparse_core_info()
    nw = info.num_cores * info.num_subcores
    nsub_a = nrow_a // nw
    nsub_b = nrow_b // nw
    mesh = plsc.VectorSubcoreMesh(core_axis_name="c", subcore_axis_name="s")

    @functools.partial(
        pl.kernel, mesh=mesh,
        out_type=[jax.ShapeDtypeStruct((nrow_a * lw, D), jnp.float32),
                  jax.ShapeDtypeStruct((nrow_b * lw, D), jnp.float32)],
        scratch_types=[
            pltpu.VMEM((nsub_a, lw), jnp.int32),
            pltpu.VMEM((nsub_b, lw), jnp.int32),
            pltpu.VMEM((2, lw, D), jnp.float32),
            pltpu.SemaphoreType.DMA,
        ],
    )
    def k(ia_hbm, ta_hbm, ib_hbm, tb_hbm, oa_hbm, ob_hbm,
          iva, ivb, rows_v, sem):
        wid = lax.axis_index("s") * info.num_cores + lax.axis_index("c")

        def run(idx_hbm, tab_hbm, out_hbm, idx_v, nsub, per_b, tab_per_b):
            rbase = wid * nsub
            pltpu.sync_copy(idx_hbm.at[pl.ds(rbase, nsub)], idx_v)
            off = rbase * lw // per_b * tab_per_b
            for r in range(nsub):
                for c in range(lw // 16):
                    sl = (r, pl.ds(c * 16, 16))
                    idx_v[sl] = idx_v[sl] + off
            copies = []
            for r in range(nsub):
                copies.append(pltpu.async_copy(
                    tab_hbm.at[idx_v.at[r]], rows_v.at[r % 2], sem))
                if r >= 1:
                    copies[r - 1].wait()
                    pltpu.sync_copy(
                        rows_v.at[(r - 1) % 2],
                        out_hbm.at[pl.ds((rbase + r - 1) * lw, lw)])
            copies[nsub - 1].wait()
            pltpu.sync_copy(rows_v.at[(nsub - 1) % 2],
                            out_hbm.at[pl.ds((rbase + nsub - 1) * lw, lw)])

        run(ia_hbm, ta_hbm, oa_hbm, iva, nsub_a, per_b_a, tab_per_b_a)
        run(ib_hbm, tb_hbm, ob_hbm, ivb, nsub_b, per_b_b, tab_per_b_b)

    return k(idx2d_a, table_a, idx2d_b, table_b)


def _selectors(f32):
    pq_col = jax.lax.broadcasted_iota(jnp.int32, (A, A * A), 1)
    pq_row = jax.lax.broadcasted_iota(jnp.int32, (A, A * A), 0)
    Psel = jnp.where(pq_col // A == pq_row, 1.0, 0.0).astype(f32)
    Qsel = jnp.where(pq_col % A == pq_row, 1.0, 0.0).astype(f32)
    pr_row = jax.lax.broadcasted_iota(jnp.int32, (A, len(PAIRS)), 0)
    pr_col = jax.lax.broadcasted_iota(jnp.int32, (A, len(PAIRS)), 1)
    E1 = jnp.zeros((A, len(PAIRS)), f32)
    E2 = jnp.zeros((A, len(PAIRS)), f32)
    for j, (p, q) in enumerate(PAIRS):
        hit = pr_col == j
        E1 = jnp.where(hit & (pr_row == p), 1.0, E1).astype(f32)
        E2 = jnp.where(hit & (pr_row == q), 1.0, E2).astype(f32)
    return Psel, Qsel, E1, E2


def _sel_dists(cols_a, cols_b, sel_a, sel_b):
    acc = None
    for ca, cb in zip(cols_a, cols_b):
        diff = _dot(ca, sel_a) - _dot(cb, sel_b)
        acc = diff * diff if acc is None else acc + diff * diff
    return jnp.sqrt(acc + 1e-12)


def _pick(arr, stride, offset):
    return jnp.concatenate(
        [arr[:, p * stride + offset:p * stride + offset + 1]
         for p in range(A)], axis=1)


def _a2(xyzs_ref, xyzst_ref, nd_ref, ni_ref, *, ns):
    xyz_s = xyzs_ref[0]
    xyz_sT = xyzst_ref[0]
    iota_s = jax.lax.broadcasted_iota(jnp.int32, (ns, ns), 1)
    d2 = None
    for c in range(3):
        diff = xyz_s[:, c:c + 1] - xyz_sT[c:c + 1, :]
        d2 = d2 + diff * diff if c else diff * diff
    dwork = d2
    nd_cols, ni_cols = [], []
    for _ in range(K):
        m = jnp.min(dwork, axis=1, keepdims=True)
        cand = jnp.where(dwork <= m, iota_s, jnp.int32(ns))
        arg = jnp.min(cand, axis=1, keepdims=True)
        nd_cols.append(m)
        ni_cols.append(arg)
        dwork = jnp.where(iota_s == arg, _INF, dwork)
    nd_ref[0] = jnp.sqrt(jnp.concatenate(nd_cols, axis=1) + 1e-12)
    ni_ref[0] = jnp.concatenate(ni_cols, axis=1)


def _a3(ac2_ref, nd_ref, ac_ref, tr2_ref, agi_ref, *, ns):
    f32 = jnp.float32
    ac2 = ac2_ref[0]
    nd = nd_ref[0]
    acx = _pick(ac2, 128, 0)
    acy = _pick(ac2, 128, 1)
    acz = _pick(ac2, 128, 2)
    samp_g = _pick(ac2, 128, 77)
    _, _, E1, E2 = _selectors(f32)
    subs = _sel_dists((acx, acy, acz), (acx, acy, acz), E1, E2)
    ia = jnp.concatenate([nd[:, 1:A], subs], axis=1)
    ac_ref[0] = jnp.concatenate(
        [acx, acy, acz, samp_g, ia, jnp.zeros((ns, 2), f32)], axis=1)
    tr2_ref[0] = jnp.concatenate(
        [acx, acx, acx, acx, acy, acy, acy, acy, acz, acz, acz, acz,
         nd[:, 1:A], subs, jnp.zeros((ns, 74), f32)], axis=1)
    agi_ref[0] = samp_g.astype(jnp.int32)


def _a45(ac_ref, nac_ref, f_ref, xbnb_ref, xbag_ref, rel_ref, adf_ref, *, ns):
    f32 = jnp.float32
    ac = ac_ref[0]
    acx, acy, acz = ac[:, 0:4], ac[:, 4:8], ac[:, 8:12]
    ia = ac[:, 16:22]
    nac_all = nac_ref[0]
    m_col = jax.lax.broadcasted_iota(jnp.int32, (A, A * A), 1)
    m_row = jax.lax.broadcasted_iota(jnp.int32, (A, A * A), 0)
    RepP = jnp.where(m_col // A == m_row, 1.0, 0.0).astype(f32)
    acp48 = jnp.concatenate(
        [_dot(acx, RepP), _dot(acy, RepP), _dot(acz, RepP)], axis=1)
    rel_k = []
    for k in range(K):
        blk = nac_all[:, k * 128:k * 128 + 48]
        diff = acp48 - blk
        sq = diff * diff
        d2 = sq[:, 0:16] + sq[:, 16:32] + sq[:, 32:48]
        inter_k = jnp.sqrt(d2 + 1e-12)
        neigh = nac_all[:, k * 128 + 48:k * 128 + 54]
        rel_k.append(jnp.concatenate([ia, neigh, inter_k], axis=1))
    rel_ref[0] = jnp.concatenate(rel_k, axis=1)
    ib_s = f_ref[0][:, 67:73]
    cols = []
    for ref in (xbnb_ref, xbag_ref):
        xb = ref[0][0]
        cols.append((_pick(xb, 128, 0), _pick(xb, 128, 1), _pick(xb, 128, 2)))
    Psel, Qsel, _, _ = _selectors(f32)
    inter_fin = _sel_dists(cols[0], cols[1], Psel, Qsel)
    adf_ref[0] = jnp.concatenate([ib_s, ia, inter_fin], axis=1)


def _b(fea_s_ref, adf_ref, Ww_ref, bw_ref, gw_ref, betw_ref, Wb_ref, bb_ref,
       gb_ref, betb_ref, Wo_ref, bo_ref, go_ref, beto_ref, out_ref):
    fea_s = fea_s_ref[...]
    adf = adf_ref[...]
    dflt = jax.lax.Precision.DEFAULT

    def bn(x, g, b):
        m = jnp.mean(x, axis=0, keepdims=True)
        v = jnp.mean((x - m) ** 2, axis=0, keepdims=True)
        return (x - m) / jnp.sqrt(v + 1e-5) * g + b

    def leaky(x):
        return jnp.where(x >= 0, x, 0.2 * x)

    w = bn(_dot(adf, Ww_ref[...], dflt) + bw_ref[...], gw_ref[...],
           betw_ref[...])
    bi = bn(_dot(adf, Wb_ref[...], dflt) + bb_ref[...], gb_ref[...],
            betb_ref[...])
    fea = leaky(fea_s * w + bi)
    cat = jnp.concatenate([fea, adf], axis=1)
    z = _dot(cat, Wo_ref[...], dflt) + bo_ref[...]
    out_ref[...] = leaky(bn(z, go_ref[...], beto_ref[...]))


def _call(body, grid_b, in_arrays, in_blocks, out_blocks, out_shapes, **kw):
    return pl.pallas_call(
        functools.partial(body, **kw),
        grid=(grid_b,),
        in_specs=[pl.BlockSpec(s, lambda i, r=len(s) - 1: (i,) + (0,) * r)
                  for s in in_blocks],
        out_specs=[pl.BlockSpec(s, lambda i, r=len(s) - 1: (i,) + (0,) * r)
                   for s in out_blocks],
        out_shape=[jax.ShapeDtypeStruct(s, dt) for s, dt in out_shapes],
    )(*in_arrays)


def kernel(xyz, feature, raw_neighbors_feature, neighbors_idx_before,
           sample_indexes, Ww, bw, gw, betw, Wb, bb, gb, betb, Wo, bo, go,
           beto):
    b, n, _ = xyz.shape
    ns = sample_indexes.shape[1]
    f32 = jnp.float32
    i32 = jnp.int32

    intra_before = raw_neighbors_feature[:, :, 0, :INTRA]
    nb4f = neighbors_idx_before[:, :, :A].astype(f32)
    row_id = jax.lax.broadcasted_iota(f32, (b, n, 1), 1)
    t1 = jnp.concatenate(
        [xyz, feature, intra_before, nb4f, row_id,
         jnp.zeros((b, n, 50), f32)], axis=2)
    samp = sample_indexes.astype(i32)

    F = _sc_gather(samp.reshape(-1, 128), t1.reshape(b * n, 128),
                   per_b=ns, tab_per_b=n).reshape(b, ns, 128)
    xyz_s = F[:, :, 0:3]
    fea_s = F[:, :, 3:67]
    xyz_sT = jnp.swapaxes(xyz_s, 1, 2)

    nd, ni = _call(_a2, b, (xyz_s, xyz_sT),
                   [(1, ns, 3), (1, 3, ns)], [(1, ns, K), (1, ns, K)],
                   [((b, ns, K), f32), ((b, ns, K), i32)], ns=ns)

    ac2 = _sc_gather(ni[:, :, :A].reshape(-1, 128),
                     F.reshape(b * ns, 128),
                     per_b=ns * A, tab_per_b=ns).reshape(b, ns, A * 128)

    ac, t_r2, ag_i = _call(
        _a3, b, (ac2, nd),
        [(1, ns, A * 128), (1, ns, K)],
        [(1, ns, 24), (1, ns, 128), (1, ns, A)],
        [((b, ns, 24), f32), ((b, ns, 128), f32), ((b, ns, A), i32)], ns=ns)

    nb_i = F[:, :, 73:77].astype(i32)
    idx_ba = jnp.concatenate(
        [nb_i.reshape(b, ns * A), ag_i.reshape(b, ns * A)], axis=1)
    nac, xb = _sc_gather2(
        ni.reshape(-1, 128), t_r2.reshape(b * ns, 128), ns * K, ns,
        idx_ba.reshape(-1, 128), t1.reshape(b * n, 128), 2 * ns * A, n)
    nac = nac.reshape(b, ns, K * 128)
    xb = xb.reshape(b, 2, ns, A * 128)

    rel, adf = pl.pallas_call(
        functools.partial(_a45, ns=ns),
        grid=(b,),
        in_specs=[
            pl.BlockSpec((1, ns, 24), lambda i: (i, 0, 0)),
            pl.BlockSpec((1, ns, K * 128), lambda i: (i, 0, 0)),
            pl.BlockSpec((1, ns, 128), lambda i: (i, 0, 0)),
            pl.BlockSpec((1, 1, ns, A * 128), lambda i: (i, 0, 0, 0)),
            pl.BlockSpec((1, 1, ns, A * 128), lambda i: (i, 1, 0, 0)),
        ],
        out_specs=[
            pl.BlockSpec((1, ns, K * CH), lambda i: (i, 0, 0)),
            pl.BlockSpec((1, ns, CH), lambda i: (i, 0, 0)),
        ],
        out_shape=[jax.ShapeDtypeStruct((b, ns, K * CH), f32),
                   jax.ShapeDtypeStruct((b, ns, CH), f32)],
    )(ac, nac, F, xb, xb)

    out = pl.pallas_call(
        _b, out_shape=jax.ShapeDtypeStruct((b * ns, 128), f32),
    )(fea_s.reshape(b * ns, 64), adf.reshape(b * ns, CH),
      Ww, bw.reshape(1, -1), gw.reshape(1, -1), betw.reshape(1, -1),
      Wb, bb.reshape(1, -1), gb.reshape(1, -1), betb.reshape(1, -1),
      Wo, bo.reshape(1, -1), go.reshape(1, -1), beto.reshape(1, -1))

    return (xyz_s, out.reshape(b, ns, 128), rel.reshape(b, ns, K, CH), ni)
